# baseline (speedup 1.0000x reference)
import functools

import jax
import jax.numpy as jnp
from jax import lax
from jax.experimental import pallas as pl
from jax.experimental.pallas import tpu as pltpu
from jax.experimental.pallas import tpu_sc as plsc

_info = plsc.get_sparse_core_info()
_NC, _NS = _info.num_cores, _info.num_subcores
_NW = _NC * _NS

_DEPTH = 4


def _make_gather(B: int, D: int, chunk: int):
    assert B % (_NW * chunk) == 0
    b_per_w = B // _NW
    n_chunks = b_per_w // chunk
    assert n_chunks >= 2 * _DEPTH and (n_chunks - _DEPTH) % _DEPTH == 0
    mesh = plsc.VectorSubcoreMesh(core_axis_name="c", subcore_axis_name="s")

    scratch = (
        [pltpu.VMEM((chunk,), jnp.int32) for _ in range(_DEPTH)]
        + [pltpu.VMEM((chunk, D), jnp.float32) for _ in range(_DEPTH)]
        + [pltpu.SemaphoreType.DMA] * (3 * _DEPTH)
    )

    @functools.partial(
        pl.kernel,
        mesh=mesh,
        out_type=jax.ShapeDtypeStruct((B, D), jnp.float32),
        compiler_params=pltpu.CompilerParams(use_tc_tiling_on_sc=False),
        scratch_types=scratch,
    )
    def gather(idx_hbm, table_hbm, out_hbm, *bufs):
        idx_v = bufs[:_DEPTH]
        rows_v = bufs[_DEPTH:2 * _DEPTH]
        gsem = bufs[2 * _DEPTH:3 * _DEPTH]
        osem = bufs[3 * _DEPTH:4 * _DEPTH]
        isem = bufs[4 * _DEPTH:5 * _DEPTH]
        wid = lax.axis_index("s") * _NC + lax.axis_index("c")
        base = wid * b_per_w

        for s in range(_DEPTH):
            pltpu.sync_copy(idx_hbm.at[pl.ds(base + s * chunk, chunk)], idx_v[s])
            pltpu.async_copy(table_hbm.at[idx_v[s]], rows_v[s], gsem[s])

        def step(c, s):
            off = base + c * chunk
            off2 = off + _DEPTH * chunk
            pltpu.make_async_copy(table_hbm.at[idx_v[s]], rows_v[s], gsem[s]).wait()
            pltpu.async_copy(rows_v[s], out_hbm.at[pl.ds(off, chunk)], osem[s])
            pltpu.async_copy(idx_hbm.at[pl.ds(off2, chunk)], idx_v[s], isem[s])
            pltpu.make_async_copy(rows_v[s], out_hbm.at[pl.ds(off, chunk)], osem[s]).wait()
            pltpu.make_async_copy(idx_hbm.at[pl.ds(off2, chunk)], idx_v[s], isem[s]).wait()
            pltpu.async_copy(table_hbm.at[idx_v[s]], rows_v[s], gsem[s])

        def body(i, carry):
            for s in range(_DEPTH):
                step(_DEPTH * i + s, s)
            return carry

        lax.fori_loop(0, (n_chunks - _DEPTH) // _DEPTH, body, 0)

        for c in range(n_chunks - _DEPTH, n_chunks):
            s = c % _DEPTH
            off = base + c * chunk
            pltpu.make_async_copy(table_hbm.at[idx_v[s]], rows_v[s], gsem[s]).wait()
            pltpu.async_copy(rows_v[s], out_hbm.at[pl.ds(off, chunk)], osem[s])
        for c in range(n_chunks - _DEPTH, n_chunks):
            s = c % _DEPTH
            off = base + c * chunk
            pltpu.make_async_copy(rows_v[s], out_hbm.at[pl.ds(off, chunk)], osem[s]).wait()

    return gather


def kernel(input, weight):
    b, s = input.shape
    vocab, d = weight.shape
    flat_idx = input.reshape(b * s)
    out = _make_gather(b * s, d, 800)(flat_idx, weight)
    return out.reshape(b, s, d)
